# baseline (speedup 1.0000x reference)
import jax
import jax.numpy as jnp
from jax import lax
from jax.experimental import pallas as pl
from jax.experimental.pallas import tpu as pltpu
from jax.experimental.pallas import tpu_sc as plsc

_RULE_V = 1000
_TOK_V = 100000
_REF_L = 200

_NW = 16

_TOK_CHUNK = 6400
_TAIL_BASE = 15 * _TOK_CHUNK
_TAIL_ALN = 3968
_TOK_RAG_BASE = _TAIL_BASE + _TAIL_ALN
_TOK_RAG = _TOK_V - _TOK_RAG_BASE

_RULE_ALN = 896
_RULE_RAG = _RULE_V - _RULE_ALN
_REF_ALN = 128
_REF_RAG = _REF_L - _REF_ALN


def _body(len_hbm, rule_hbm, token_hbm, ref_hbm,
          out_rule, out_tok, out_ref,
          len_v, idx_v, tok_buf, tail_buf, rag_buf, rule_buf, rule_rag_buf,
          ref_buf, ref_rag_buf, sem):
    wid = lax.axis_index("s")
    pltpu.sync_copy(len_hbm, len_v.at[pl.ds(0, 1)])
    vec = len_v[...] - 1
    idx_v[...] = vec
    idx1 = idx_v.at[pl.ds(0, 1)]

    @pl.when(wid < _NW - 1)
    def _():
        base = pl.multiple_of(wid * _TOK_CHUNK, 128)
        pltpu.async_copy(
            token_hbm.at[idx1, pl.ds(base, _TOK_CHUNK)], tok_buf, sem
        ).wait()
        pltpu.sync_copy(tok_buf.at[0], out_tok.at[pl.ds(base, _TOK_CHUNK)])

    @pl.when(wid == _NW - 1)
    def _():
        row = vec[0]
        row8 = pl.multiple_of((row // 8) * 8, 8)
        rsub = row - row8
        pltpu.async_copy(
            token_hbm.at[idx1, pl.ds(_TAIL_BASE, _TAIL_ALN)], tail_buf, sem
        ).wait()
        pltpu.sync_copy(tail_buf.at[0], out_tok.at[pl.ds(_TAIL_BASE, _TAIL_ALN)])
        pltpu.async_copy(
            rule_hbm.at[idx1, pl.ds(0, _RULE_ALN)], rule_buf, sem
        ).wait()
        pltpu.sync_copy(rule_buf.at[0], out_rule.at[pl.ds(0, _RULE_ALN)])
        pltpu.async_copy(
            ref_hbm.at[idx1, pl.ds(0, _REF_ALN)], ref_buf, sem
        ).wait()
        pltpu.sync_copy(ref_buf.at[0], out_ref.at[pl.ds(0, _REF_ALN)])
        pltpu.sync_copy(
            token_hbm.at[pl.ds(row8, 8), pl.ds(_TOK_RAG_BASE, _TOK_RAG)],
            rag_buf)
        pltpu.sync_copy(rag_buf.at[rsub],
                        out_tok.at[pl.ds(_TOK_RAG_BASE, _TOK_RAG)])
        pltpu.sync_copy(
            rule_hbm.at[pl.ds(row8, 8), pl.ds(_RULE_ALN, _RULE_RAG)],
            rule_rag_buf)
        pltpu.sync_copy(rule_rag_buf.at[rsub],
                        out_rule.at[pl.ds(_RULE_ALN, _RULE_RAG)])
        pltpu.sync_copy(
            ref_hbm.at[pl.ds(row8, 8), pl.ds(_REF_ALN, _REF_RAG)],
            ref_rag_buf)
        pltpu.sync_copy(ref_rag_buf.at[rsub],
                        out_ref.at[pl.ds(_REF_ALN, _REF_RAG)])


@jax.jit
def _select_rows(length, rule_prob, token_prob, reference_prob):
    mesh = plsc.VectorSubcoreMesh(
        core_axis_name="c", subcore_axis_name="s", num_cores=1)
    return pl.kernel(
        _body,
        out_type=[
            jax.ShapeDtypeStruct((_RULE_V,), jnp.float32),
            jax.ShapeDtypeStruct((_TOK_V,), jnp.float32),
            jax.ShapeDtypeStruct((_REF_L,), jnp.float32),
        ],
        mesh=mesh,
        scratch_types=[
            pltpu.VMEM((16,), jnp.int32),
            pltpu.VMEM((16,), jnp.int32),
            pltpu.VMEM((1, _TOK_CHUNK), jnp.float32),
            pltpu.VMEM((1, _TAIL_ALN), jnp.float32),
            pltpu.VMEM((8, _TOK_RAG), jnp.float32),
            pltpu.VMEM((1, _RULE_ALN), jnp.float32),
            pltpu.VMEM((8, _RULE_RAG), jnp.float32),
            pltpu.VMEM((1, _REF_ALN), jnp.float32),
            pltpu.VMEM((8, _REF_RAG), jnp.float32),
            pltpu.SemaphoreType.DMA,
        ],
        compiler_params=pltpu.CompilerParams(
            use_tc_tiling_on_sc=True, skip_device_barrier=True),
    )(length, rule_prob, token_prob, reference_prob)


def kernel(rule_prob, token_prob, reference_prob, length):
    rule_row, tok_row, ref_row = _select_rows(
        length, rule_prob, token_prob, reference_prob)
    return (rule_row, tok_row, ref_row)


# speedup vs baseline: 1.1323x; 1.1323x over previous
import jax
import jax.numpy as jnp
from jax import lax
from jax.experimental import pallas as pl
from jax.experimental.pallas import tpu as pltpu
from jax.experimental.pallas import tpu_sc as plsc

_RULE_V = 1000
_TOK_V = 100000
_REF_L = 200

_NW = 16

_TOK_CHUNK = 6400
_TAIL_BASE = 15 * _TOK_CHUNK
_TAIL_ALN = 3968
_TOK_RAG_BASE = _TAIL_BASE + _TAIL_ALN
_TOK_RAG = _TOK_V - _TOK_RAG_BASE

_RULE_ALN = 896
_RULE_RAG = _RULE_V - _RULE_ALN
_REF_ALN = 128
_REF_RAG = _REF_L - _REF_ALN


def _body(len_hbm, rule_hbm, token_hbm, ref_hbm,
          out_rule, out_tok, out_ref,
          len_v, idx_v, tok_buf, tail_buf, rag_buf, rule_buf, rule_rag_buf,
          ref_buf, ref_rag_buf, sem, sem2, sem3, sem4, sem5, sem6,
          wsem1, wsem2, wsem3, wsem4, wsem5, wsem6):
    wid = lax.axis_index("s")
    pltpu.sync_copy(len_hbm, len_v.at[pl.ds(0, 1)])
    vec = len_v[...] - 1
    idx_v[...] = vec
    idx1 = idx_v.at[pl.ds(0, 1)]
    row = vec[0]
    row8 = pl.multiple_of((row // 8) * 8, 8)
    rsub = row - row8

    @pl.when(wid < _NW - 1)
    def _():
        base = pl.multiple_of(wid * _TOK_CHUNK, 128)
        pltpu.async_copy(
            token_hbm.at[idx1, pl.ds(base, _TOK_CHUNK)], tok_buf, sem
        ).wait()
        pltpu.sync_copy(tok_buf.at[0], out_tok.at[pl.ds(base, _TOK_CHUNK)])

    @pl.when(wid == _NW - 1)
    def _():
        g1 = pltpu.async_copy(
            token_hbm.at[idx1, pl.ds(_TAIL_BASE, _TAIL_ALN)], tail_buf, sem)
        g2 = pltpu.async_copy(
            token_hbm.at[pl.ds(row8, 8), pl.ds(_TOK_RAG_BASE, _TOK_RAG)],
            rag_buf, sem2)
        g3 = pltpu.async_copy(
            rule_hbm.at[idx1, pl.ds(0, _RULE_ALN)], rule_buf, sem3)
        g4 = pltpu.async_copy(
            rule_hbm.at[pl.ds(row8, 8), pl.ds(_RULE_ALN, _RULE_RAG)],
            rule_rag_buf, sem4)
        g5 = pltpu.async_copy(
            ref_hbm.at[idx1, pl.ds(0, _REF_ALN)], ref_buf, sem5)
        g6 = pltpu.async_copy(
            ref_hbm.at[pl.ds(row8, 8), pl.ds(_REF_ALN, _REF_RAG)],
            ref_rag_buf, sem6)
        g1.wait()
        w1 = pltpu.make_async_copy(
            tail_buf.at[0], out_tok.at[pl.ds(_TAIL_BASE, _TAIL_ALN)], wsem1)
        w1.start()
        g2.wait()
        w2 = pltpu.make_async_copy(
            rag_buf.at[rsub], out_tok.at[pl.ds(_TOK_RAG_BASE, _TOK_RAG)], wsem2)
        w2.start()
        g3.wait()
        w3 = pltpu.make_async_copy(
            rule_buf.at[0], out_rule.at[pl.ds(0, _RULE_ALN)], wsem3)
        w3.start()
        g4.wait()
        w4 = pltpu.make_async_copy(
            rule_rag_buf.at[rsub], out_rule.at[pl.ds(_RULE_ALN, _RULE_RAG)],
            wsem4)
        w4.start()
        g5.wait()
        w5 = pltpu.make_async_copy(
            ref_buf.at[0], out_ref.at[pl.ds(0, _REF_ALN)], wsem5)
        w5.start()
        g6.wait()
        w6 = pltpu.make_async_copy(
            ref_rag_buf.at[rsub], out_ref.at[pl.ds(_REF_ALN, _REF_RAG)], wsem6)
        w6.start()
        w1.wait()
        w2.wait()
        w3.wait()
        w4.wait()
        w5.wait()
        w6.wait()


@jax.jit
def _select_rows(length, rule_prob, token_prob, reference_prob):
    mesh = plsc.VectorSubcoreMesh(
        core_axis_name="c", subcore_axis_name="s", num_cores=1)
    return pl.kernel(
        _body,
        out_type=[
            jax.ShapeDtypeStruct((_RULE_V,), jnp.float32),
            jax.ShapeDtypeStruct((_TOK_V,), jnp.float32),
            jax.ShapeDtypeStruct((_REF_L,), jnp.float32),
        ],
        mesh=mesh,
        scratch_types=[
            pltpu.VMEM((16,), jnp.int32),
            pltpu.VMEM((16,), jnp.int32),
            pltpu.VMEM((1, _TOK_CHUNK), jnp.float32),
            pltpu.VMEM((1, _TAIL_ALN), jnp.float32),
            pltpu.VMEM((8, _TOK_RAG), jnp.float32),
            pltpu.VMEM((1, _RULE_ALN), jnp.float32),
            pltpu.VMEM((8, _RULE_RAG), jnp.float32),
            pltpu.VMEM((1, _REF_ALN), jnp.float32),
            pltpu.VMEM((8, _REF_RAG), jnp.float32),
        ] + [pltpu.SemaphoreType.DMA] * 12,
        compiler_params=pltpu.CompilerParams(
            use_tc_tiling_on_sc=True, skip_device_barrier=True),
    )(length, rule_prob, token_prob, reference_prob)


def kernel(rule_prob, token_prob, reference_prob, length):
    rule_row, tok_row, ref_row = _select_rows(
        length, rule_prob, token_prob, reference_prob)
    return (rule_row, tok_row, ref_row)
